# Initial kernel scaffold; baseline (speedup 1.0000x reference)
import jax
import jax.numpy as jnp
from jax.experimental import pallas as pl


def kernel(x, edge_index, edge_attr, W1, b1, W2, b2, W3, b3, g1, be1, g2, be2):
    raise NotImplementedError("write your pallas kernel here")


# speedup vs baseline: 7.3702x; 7.3702x over previous
import functools

import jax
import jax.numpy as jnp
from jax import lax
from jax.experimental import pallas as pl
from jax.experimental.pallas import tpu as pltpu
from jax.experimental.pallas import tpu_sc as plsc

N = 10000
E = 320000
D = 128

NC = 2
NS = 16
NW = NC * NS
EPW = E // NW
CHUNK = 80
NCHUNK = EPW // CHUNK
N_PAD = 10240
SLICE = N_PAD // NS

_MESH = plsc.VectorSubcoreMesh(core_axis_name="c", subcore_axis_name="s")


@functools.partial(
    pl.kernel,
    out_type=jax.ShapeDtypeStruct((NC * N_PAD,), jnp.float32),
    mesh=_MESH,
    scratch_types=[
        pltpu.VMEM((CHUNK,), jnp.int32),
        pltpu.VMEM((CHUNK,), jnp.float32),
        pltpu.VMEM((SLICE,), jnp.float32),
        pltpu.VMEM_SHARED((N_PAD,), jnp.float32),
    ],
)
def _deg_kernel(dst_hbm, out_hbm, idx_v, ones_v, stage_v, acc_sh):
    c = lax.axis_index("c")
    s = lax.axis_index("s")
    wid = s * NC + c

    def zero_body(i, _):
        stage_v[pl.ds(i * 16, 16)] = jnp.zeros((16,), jnp.float32)
        return 0

    lax.fori_loop(0, SLICE // 16, zero_body, 0)
    for j in range(CHUNK // 16):
        ones_v[pl.ds(j * 16, 16)] = jnp.ones((16,), jnp.float32)

    pltpu.sync_copy(stage_v, acc_sh.at[pl.ds(s * SLICE, SLICE)])
    plsc.subcore_barrier()

    base = wid * EPW

    def body(i, _):
        pltpu.sync_copy(dst_hbm.at[pl.ds(base + i * CHUNK, CHUNK)], idx_v)
        pltpu.sync_copy(ones_v, acc_sh.at[idx_v], add=True)
        return 0

    lax.fori_loop(0, NCHUNK, body, 0)
    plsc.subcore_barrier()

    pltpu.sync_copy(acc_sh.at[pl.ds(s * SLICE, SLICE)], stage_v)
    pltpu.sync_copy(stage_v, out_hbm.at[pl.ds(c * N_PAD + s * SLICE, SLICE)])


HALF = N // 2
ACC_R = 5008
PT_S = 312
PT_L = ACC_R - 15 * PT_S
OFF_L = 15 * PT_S
EPT = E // NS
NCH = EPT // CHUNK


@functools.partial(
    pl.kernel,
    out_type=jax.ShapeDtypeStruct((NC * ACC_R, D), jnp.float32),
    mesh=_MESH,
    scratch_types=[
        pltpu.VMEM((CHUNK,), jnp.int32),
        pltpu.VMEM((CHUNK,), jnp.int32),
        pltpu.VMEM((CHUNK, D), jnp.float32),
        pltpu.VMEM((PT_L, D), jnp.float32),
        pltpu.VMEM_SHARED((ACC_R, D), jnp.float32),
        pltpu.SemaphoreType.DMA,
    ],
)
def _agg_kernel(hn_hbm, src_hbm, dst_hbm, out_hbm,
                sidx_v, didx_v, rows_v, stage_v, acc_sh, sem):
    c = lax.axis_index("c")
    s = lax.axis_index("s")

    def zero_row(i, _):
        for j in range(D // 16):
            stage_v[i, pl.ds(j * 16, 16)] = jnp.zeros((16,), jnp.float32)
        return 0

    lax.fori_loop(0, PT_L, zero_row, 0)

    @pl.when(s < 15)
    def _():
        pltpu.sync_copy(stage_v.at[pl.ds(0, PT_S)],
                        acc_sh.at[pl.ds(s * PT_S, PT_S)])

    @pl.when(s == 15)
    def _():
        pltpu.sync_copy(stage_v, acc_sh.at[pl.ds(OFF_L, PT_L)])

    plsc.subcore_barrier()

    base = s * EPT
    lo = c * HALF

    def chunk_body(i, _):
        pltpu.sync_copy(src_hbm.at[pl.ds(base + i * CHUNK, CHUNK)], sidx_v)
        cp = pltpu.async_copy(hn_hbm.at[sidx_v], rows_v, sem)
        pltpu.sync_copy(dst_hbm.at[pl.ds(base + i * CHUNK, CHUNK)], didx_v)

        def fix(j, _):
            d = didx_v[pl.ds(j * 16, 16)]
            rel = d - lo
            ok = (rel >= 0) & (rel < HALF)
            didx_v[pl.ds(j * 16, 16)] = jnp.where(ok, rel, HALF)
            return 0

        lax.fori_loop(0, CHUNK // 16, fix, 0)
        cp.wait()
        pltpu.sync_copy(rows_v, acc_sh.at[didx_v], add=True)
        return 0

    lax.fori_loop(0, NCH, chunk_body, 0)
    plsc.subcore_barrier()

    @pl.when(s < 15)
    def _():
        pltpu.sync_copy(acc_sh.at[pl.ds(s * PT_S, PT_S)],
                        stage_v.at[pl.ds(0, PT_S)])
        pltpu.sync_copy(stage_v.at[pl.ds(0, PT_S)],
                        out_hbm.at[pl.ds(c * ACC_R + s * PT_S, PT_S)])

    @pl.when(s == 15)
    def _():
        pltpu.sync_copy(acc_sh.at[pl.ds(OFF_L, PT_L)], stage_v)
        pltpu.sync_copy(stage_v, out_hbm.at[pl.ds(c * ACC_R + OFF_L, PT_L)])


def _t_first_body(x_ref, w_ref, dis_ref, out_ref):
    h = jnp.dot(x_ref[...], w_ref[...], preferred_element_type=jnp.float32)
    out_ref[...] = h * dis_ref[...]


def _t_layer_body(agg_ref, hn_ref, dis_ref, b_ref, g_ref, be_ref, w_ref,
                  t_ref, hn2_ref):
    a = jnp.concatenate(
        [agg_ref[pl.ds(0, HALF), :], agg_ref[pl.ds(ACC_R, HALF), :]], axis=0)
    t = (a + hn_ref[...]) * dis_ref[...] + b_ref[...]
    t_ref[...] = t
    u = jnp.maximum(t * g_ref[...] + be_ref[...], 0.0)
    h = jnp.dot(u, w_ref[...], preferred_element_type=jnp.float32)
    hn2_ref[...] = h * dis_ref[...]


_full = jax.ShapeDtypeStruct((N, D), jnp.float32)
_t_first = pl.pallas_call(_t_first_body, out_shape=_full)
_t_layer = pl.pallas_call(_t_layer_body, out_shape=(_full, _full))


def kernel(x, edge_index, edge_attr, W1, b1, W2, b2, W3, b3, g1, be1, g2, be2):
    src = edge_index[0]
    dst = edge_index[1]

    deg_parts = _deg_kernel(dst).reshape(NC, N_PAD)
    deg = deg_parts[0, :N] + deg_parts[1, :N] + 1.0
    dis = lax.rsqrt(deg)[:, None]

    bn = 1.0 / jnp.sqrt(jnp.float32(1.0 + 1e-5))
    ones = jnp.ones((D,), jnp.float32)
    zeros = jnp.zeros((D,), jnp.float32)
    Ws = jnp.stack([W2, W3, W1])
    bs = jnp.stack([b1, b2, b3])
    gs = jnp.stack([g1 * bn, g2 * bn, ones])
    bes = jnp.stack([be1, be2, zeros])

    hn1 = _t_first(x, W1, dis)

    def layer(hn, p):
        Wl, bl, gl, bel = p
        agg = _agg_kernel(hn, src, dst)
        t, hn_next = _t_layer(agg, hn, dis, bl[None, :], gl[None, :],
                              bel[None, :], Wl)
        return hn_next, t

    _, ts = lax.scan(layer, hn1, (Ws, bs, gs, bes))
    return (ts[-1], edge_attr)
